# baseline (speedup 1.0000x reference)
import functools
import math

import jax
import jax.numpy as jnp
from jax import lax
from jax.experimental import pallas as pl
from jax.experimental.pallas import tpu as pltpu
from jax.experimental.pallas import tpu_sc as plsc

_HALF_LOG_2PI = 0.5 * math.log(2.0 * math.pi)

_L = 64
_K = 128
_B = 4096
_NC = 2
_NS = 16
_NW = _NC * _NS
_BPW = _B // _NW
_LANES = 16


def _sc_body(zt_hbm, mut_hbm, lvt_hbm, w_hbm, m_hbm, s_hbm,
             zt_v, mut_v, lvt_v, w_v, a_v, b_v, c_v, m_v, s_v):
    wid = lax.axis_index("s") * _NC + lax.axis_index("c")
    base = wid * _BPW
    pltpu.sync_copy(zt_hbm.at[:, pl.ds(base, _BPW)], zt_v)
    pltpu.sync_copy(mut_hbm, mut_v)
    pltpu.sync_copy(lvt_hbm, lvt_v)
    pltpu.sync_copy(w_hbm, w_v)

    nkc = _K // _LANES
    nbc = _BPW // _LANES

    lw = [w_v[pl.ds(kc * _LANES, _LANES)] for kc in range(nkc)]

    def prep(l, _):
        for kc in range(nkc):
            sl = pl.ds(kc * _LANES, _LANES)
            muv = mut_v[l, sl]
            lvv = lvt_v[l, sl]
            pv = jnp.exp(-lvv)
            a_v[l, sl] = ((-_HALF_LOG_2PI) - 0.5 * lvv
                          - 0.5 * pv * muv * muv + lw[kc])
            b_v[l, sl] = pv * muv
            c_v[l, sl] = -0.5 * pv
        return _

    lax.fori_loop(0, _L, prep, 0)

    def ldim(l, _):
        zv = [zt_v[l, pl.ds(bc * _LANES, _LANES)] for bc in range(nbc)]
        z2 = [v * v for v in zv]

        def pass1(kc, m):
            sl = pl.ds(kc * _LANES, _LANES)
            av = a_v[l, sl]
            bv = b_v[l, sl]
            cv = c_v[l, sl]
            m = list(m)
            for j in range(_LANES):
                a, b, c = av[j], bv[j], cv[j]
                m = [jnp.maximum(m[bc], a + b * zv[bc] + c * z2[bc])
                     for bc in range(nbc)]
            return tuple(m)

        m0 = tuple(jnp.full((_LANES,), -1e30, jnp.float32)
                   for _x in range(nbc))
        m = lax.fori_loop(0, nkc, pass1, m0)

        def pass2(kc, s):
            sl = pl.ds(kc * _LANES, _LANES)
            av = a_v[l, sl]
            bv = b_v[l, sl]
            cv = c_v[l, sl]
            s = list(s)
            for j in range(_LANES):
                a, b, c = av[j], bv[j], cv[j]
                s = [s[bc] + jnp.exp(a + b * zv[bc] + c * z2[bc] - m[bc])
                     for bc in range(nbc)]
            return tuple(s)

        s0 = tuple(jnp.zeros((_LANES,), jnp.float32) for _x in range(nbc))
        s = lax.fori_loop(0, nkc, pass2, s0)
        for bc in range(nbc):
            m_v[l, pl.ds(bc * _LANES, _LANES)] = m[bc]
            s_v[l, pl.ds(bc * _LANES, _LANES)] = s[bc]
        return _

    lax.fori_loop(0, _L, ldim, 0)
    pltpu.sync_copy(m_v, m_hbm.at[:, pl.ds(base, _BPW)])
    pltpu.sync_copy(s_v, s_hbm.at[:, pl.ds(base, _BPW)])


_sc_call = functools.partial(
    pl.kernel,
    out_type=[jax.ShapeDtypeStruct((_L, _B), jnp.float32),
              jax.ShapeDtypeStruct((_L, _B), jnp.float32)],
    mesh=plsc.VectorSubcoreMesh(core_axis_name="c", subcore_axis_name="s"),
    scratch_types=[
        pltpu.VMEM((_L, _BPW), jnp.float32),
        pltpu.VMEM((_L, _K), jnp.float32),
        pltpu.VMEM((_L, _K), jnp.float32),
        pltpu.VMEM((_K,), jnp.float32),
        pltpu.VMEM((_L, _K), jnp.float32),
        pltpu.VMEM((_L, _K), jnp.float32),
        pltpu.VMEM((_L, _K), jnp.float32),
        pltpu.VMEM((_L, _BPW), jnp.float32),
        pltpu.VMEM((_L, _BPW), jnp.float32),
    ],
)(_sc_body)


def _epilogue_body(m_ref, s_ref, w_ref, out_ref):
    wv = w_ref[...]
    wmax = jnp.max(wv)
    logS = wmax + jnp.log(jnp.sum(jnp.exp(wv - wmax)))
    out_ref[...] = m_ref[...] + jnp.log(s_ref[...]) - logS


@jax.jit
def kernel(z, means, logvars, w):
    B, L = z.shape
    K = means.shape[0]
    zt = z.T
    w1 = w.reshape(K)
    m, s = _sc_call(zt, means.T, logvars.T, w1)
    out = pl.pallas_call(
        _epilogue_body,
        in_specs=[
            pl.BlockSpec((L, B), lambda: (0, 0)),
            pl.BlockSpec((L, B), lambda: (0, 0)),
            pl.BlockSpec((K, 1), lambda: (0, 0)),
        ],
        out_specs=pl.BlockSpec((L, B), lambda: (0, 0)),
        out_shape=jax.ShapeDtypeStruct((L, B), jnp.float32),
    )(m, s, w.reshape(K, 1))
    return out.T


# speedup vs baseline: 3.1854x; 3.1854x over previous
import functools
import math

import jax
import jax.numpy as jnp
from jax import lax
from jax.experimental import pallas as pl
from jax.experimental.pallas import tpu as pltpu
from jax.experimental.pallas import tpu_sc as plsc

_HALF_LOG_2PI = 0.5 * math.log(2.0 * math.pi)

_L = 64
_K = 128
_B = 4096
_NC = 2
_NS = 16
_NW = _NC * _NS
_LPW = _L // _NW
_LANES = 16


def _sc_body(zt_hbm, mut_hbm, lvt_hbm, w_hbm, m_hbm, s_hbm,
             zt_v, mut_v, lvt_v, w_v, abc_v, m_v, s_v, abc_sm):
    wid = lax.axis_index("s") * _NC + lax.axis_index("c")
    l0 = wid * _LPW
    pltpu.sync_copy(zt_hbm.at[pl.ds(l0, _LPW)], zt_v)
    pltpu.sync_copy(mut_hbm.at[pl.ds(l0 * _K, _LPW * _K)], mut_v)
    pltpu.sync_copy(lvt_hbm.at[pl.ds(l0 * _K, _LPW * _K)], lvt_v)
    pltpu.sync_copy(w_hbm, w_v)

    nkc = _K // _LANES

    for l in range(_LPW):
        for kc in range(nkc):
            src = pl.ds(l * _K + kc * _LANES, _LANES)
            muv = mut_v[src]
            lvv = lvt_v[src]
            lwv = w_v[pl.ds(kc * _LANES, _LANES)]
            pv = jnp.exp(-lvv)
            base = 3 * l * _K + kc * _LANES
            abc_v[pl.ds(base, _LANES)] = ((-_HALF_LOG_2PI) - 0.5 * lvv
                                          - 0.5 * pv * muv * muv + lwv)
            abc_v[pl.ds(base + _K, _LANES)] = pv * muv
            abc_v[pl.ds(base + 2 * _K, _LANES)] = -0.5 * pv
    for i in range(3 * _LPW * _K // _LANES):
        vv = abc_v[pl.ds(i * _LANES, _LANES)]
        for j in range(_LANES):
            abc_sm[i * _LANES + j] = vv[j]

    GC = 8
    nblk = _B // (GC * _LANES)

    for l in range(_LPW):
        aoff = 3 * l * _K

        def blk(i, _, l=l, aoff=aoff):
            boff = i * GC * _LANES
            zv = [zt_v[l, pl.ds(boff + bc * _LANES, _LANES)]
                  for bc in range(GC)]
            z2 = [v * v for v in zv]

            def pass1(k, m):
                a = abc_sm[aoff + k]
                b = abc_sm[aoff + _K + k]
                c = abc_sm[aoff + 2 * _K + k]
                return tuple(
                    jnp.maximum(m[bc], a + b * zv[bc] + c * z2[bc])
                    for bc in range(GC))

            m0 = tuple(jnp.full((_LANES,), -1e30, jnp.float32)
                       for _x in range(GC))
            m = lax.fori_loop(0, _K, pass1, m0, unroll=2)

            def pass2(k, s):
                a = abc_sm[aoff + k]
                b = abc_sm[aoff + _K + k]
                c = abc_sm[aoff + 2 * _K + k]
                return tuple(
                    s[bc] + jnp.exp(a + b * zv[bc] + c * z2[bc] - m[bc])
                    for bc in range(GC))

            s0 = tuple(jnp.zeros((_LANES,), jnp.float32)
                       for _x in range(GC))
            s = lax.fori_loop(0, _K, pass2, s0, unroll=2)
            for bc in range(GC):
                m_v[l, pl.ds(boff + bc * _LANES, _LANES)] = m[bc]
                s_v[l, pl.ds(boff + bc * _LANES, _LANES)] = s[bc]
            return _

        lax.fori_loop(0, nblk, blk, 0)

    pltpu.sync_copy(m_v, m_hbm.at[pl.ds(l0, _LPW)])
    pltpu.sync_copy(s_v, s_hbm.at[pl.ds(l0, _LPW)])


_sc_call = functools.partial(
    pl.kernel,
    out_type=[jax.ShapeDtypeStruct((_L, _B), jnp.float32),
              jax.ShapeDtypeStruct((_L, _B), jnp.float32)],
    mesh=plsc.VectorSubcoreMesh(core_axis_name="c", subcore_axis_name="s"),
    scratch_types=[
        pltpu.VMEM((_LPW, _B), jnp.float32),
        pltpu.VMEM((_LPW * _K,), jnp.float32),
        pltpu.VMEM((_LPW * _K,), jnp.float32),
        pltpu.VMEM((_K,), jnp.float32),
        pltpu.VMEM((3 * _LPW * _K,), jnp.float32),
        pltpu.VMEM((_LPW, _B), jnp.float32),
        pltpu.VMEM((_LPW, _B), jnp.float32),
        pltpu.SMEM((3 * _LPW * _K,), jnp.float32),
    ],
)(_sc_body)


def _epilogue_body(m_ref, s_ref, w_ref, out_ref):
    wv = w_ref[...]
    wmax = jnp.max(wv)
    logS = wmax + jnp.log(jnp.sum(jnp.exp(wv - wmax)))
    out_ref[...] = m_ref[...] + jnp.log(s_ref[...]) - logS


@jax.jit
def kernel(z, means, logvars, w):
    B, L = z.shape
    K = means.shape[0]
    zt = z.T
    w1 = w.reshape(K)
    m, s = _sc_call(zt, means.T.reshape(-1), logvars.T.reshape(-1), w1)
    out = pl.pallas_call(
        _epilogue_body,
        in_specs=[
            pl.BlockSpec((L, B), lambda: (0, 0)),
            pl.BlockSpec((L, B), lambda: (0, 0)),
            pl.BlockSpec((K, 1), lambda: (0, 0)),
        ],
        out_specs=pl.BlockSpec((L, B), lambda: (0, 0)),
        out_shape=jax.ShapeDtypeStruct((L, B), jnp.float32),
    )(m, s, w.reshape(K, 1))
    return out.T


# speedup vs baseline: 8.7119x; 2.7350x over previous
import functools
import math

import jax
import jax.numpy as jnp
from jax import lax
from jax.experimental import pallas as pl
from jax.experimental.pallas import tpu as pltpu
from jax.experimental.pallas import tpu_sc as plsc

_HALF_LOG_2PI = 0.5 * math.log(2.0 * math.pi)
_LOG2E = 1.4426950408889634
_LN2 = 0.6931471805599453

_L = 64
_K = 128
_B = 4096
_B_SC = 1024
_NC = 2
_NS = 16
_NW = _NC * _NS
_LPW = _L // _NW
_LANES = 16


def _sc_body(zt_hbm, mut_hbm, lvt_hbm, w_hbm, m_hbm, s_hbm,
             zt_v, mut_v, lvt_v, w_v, abc_v, m_v, s_v, abc_sm):
    wid = lax.axis_index("s") * _NC + lax.axis_index("c")
    l0 = wid * _LPW
    pltpu.sync_copy(zt_hbm.at[pl.ds(l0, _LPW)], zt_v)
    pltpu.sync_copy(mut_hbm.at[pl.ds(l0 * _K, _LPW * _K)], mut_v)
    pltpu.sync_copy(lvt_hbm.at[pl.ds(l0 * _K, _LPW * _K)], lvt_v)
    pltpu.sync_copy(w_hbm, w_v)

    nkc = _K // _LANES

    for l in range(_LPW):
        for kc in range(nkc):
            src = pl.ds(l * _K + kc * _LANES, _LANES)
            muv = mut_v[src]
            lvv = lvt_v[src]
            lwv = w_v[pl.ds(kc * _LANES, _LANES)]
            pv = jnp.exp(-lvv)
            base = 3 * l * _K + kc * _LANES
            abc_v[pl.ds(base, _LANES)] = ((-_HALF_LOG_2PI) - 0.5 * lvv
                                          - 0.5 * pv * muv * muv + lwv)
            abc_v[pl.ds(base + _K, _LANES)] = pv * muv
            abc_v[pl.ds(base + 2 * _K, _LANES)] = -0.5 * pv
    for i in range(3 * _LPW * _K // _LANES):
        vv = abc_v[pl.ds(i * _LANES, _LANES)]
        for j in range(_LANES):
            abc_sm[i * _LANES + j] = vv[j]

    GC = 8
    nblk = _B_SC // (GC * _LANES)

    for l in range(_LPW):
        aoff = 3 * l * _K

        def blk(i, _, l=l, aoff=aoff):
            boff = i * GC * _LANES
            zv = [zt_v[l, pl.ds(boff + bc * _LANES, _LANES)]
                  for bc in range(GC)]
            z2 = [v * v for v in zv]

            def pass1(k, m):
                a = abc_sm[aoff + k]
                b = abc_sm[aoff + _K + k]
                c = abc_sm[aoff + 2 * _K + k]
                return tuple(
                    jnp.maximum(m[bc], a + b * zv[bc] + c * z2[bc])
                    for bc in range(GC))

            m0 = tuple(jnp.full((_LANES,), -1e30, jnp.float32)
                       for _x in range(GC))
            m = lax.fori_loop(0, _K, pass1, m0, unroll=2)

            def pass2(k, s):
                a = abc_sm[aoff + k]
                b = abc_sm[aoff + _K + k]
                c = abc_sm[aoff + 2 * _K + k]
                return tuple(
                    s[bc] + jnp.exp(a + b * zv[bc] + c * z2[bc] - m[bc])
                    for bc in range(GC))

            s0 = tuple(jnp.zeros((_LANES,), jnp.float32)
                       for _x in range(GC))
            s = lax.fori_loop(0, _K, pass2, s0, unroll=2)
            for bc in range(GC):
                m_v[l, pl.ds(boff + bc * _LANES, _LANES)] = m[bc]
                s_v[l, pl.ds(boff + bc * _LANES, _LANES)] = s[bc]
            return _

        lax.fori_loop(0, nblk, blk, 0)

    pltpu.sync_copy(m_v, m_hbm.at[pl.ds(l0, _LPW)])
    pltpu.sync_copy(s_v, s_hbm.at[pl.ds(l0, _LPW)])


_sc_call = functools.partial(
    pl.kernel,
    out_type=[jax.ShapeDtypeStruct((_L, _B_SC), jnp.float32),
              jax.ShapeDtypeStruct((_L, _B_SC), jnp.float32)],
    mesh=plsc.VectorSubcoreMesh(core_axis_name="c", subcore_axis_name="s"),
    scratch_types=[
        pltpu.VMEM((_LPW, _B_SC), jnp.float32),
        pltpu.VMEM((_LPW * _K,), jnp.float32),
        pltpu.VMEM((_LPW * _K,), jnp.float32),
        pltpu.VMEM((_K,), jnp.float32),
        pltpu.VMEM((3 * _LPW * _K,), jnp.float32),
        pltpu.VMEM((_LPW, _B_SC), jnp.float32),
        pltpu.VMEM((_LPW, _B_SC), jnp.float32),
        pltpu.SMEM((3 * _LPW * _K,), jnp.float32),
    ],
)(_sc_body)


def _epilogue_body(m_ref, s_ref, w_ref, out_ref):
    wv = w_ref[...]
    wmax = jnp.max(wv)
    logS = wmax + jnp.log(jnp.sum(jnp.exp(wv - wmax)))
    out_ref[...] = m_ref[...] + jnp.log(s_ref[...]) - logS


_LG = 8
_KC = 32


def _tc_body(zt_ref, mu_ref, lv_ref, w_ref, out_ref, p3_s):
    K, L = mu_ref.shape
    Bb = zt_ref.shape[1]
    NG = L // _LG
    mu = mu_ref[...]
    lv = lv_ref[...]
    wv = w_ref[...]
    wmax = jnp.max(wv)
    logw = wv - wmax - jnp.log(jnp.sum(jnp.exp(wv - wmax)))
    prec = jnp.exp(-lv)
    a_all = _LOG2E * ((-_HALF_LOG_2PI) - 0.5 * lv
                      - 0.5 * prec * mu * mu + logw)
    b_all = _LOG2E * prec * mu
    c_all = (-0.5 * _LOG2E) * prec
    for g in range(NG):
        sl = slice(g * _LG, (g + 1) * _LG)
        p3_s[pl.ds(g, 1)] = jnp.concatenate(
            [a_all[:, sl], b_all[:, sl], c_all[:, sl]], axis=0)[None]

    def lgroup(g, _):
        pc = p3_s[pl.ds(g, 1)][0]
        for j in range(_LG):
            col = pc[:, j:j + 1]
            a = col[0:K]
            b = col[K:2 * K]
            c = col[2 * K:3 * K]
            zrow = zt_ref[pl.ds(g * _LG + j, 1), :]
            z2 = zrow * zrow
            m_run = None
            s_run = None
            for kc in range(K // _KC):
                ks = slice(kc * _KC, (kc + 1) * _KC)
                t2c = a[ks] + b[ks] * zrow + c[ks] * z2
                t3 = t2c.reshape(_KC // 8, 8, Bb)
                mc = jnp.max(t3, axis=0)
                sc = jnp.sum(jnp.exp2(t3 - mc[None]), axis=0)
                if m_run is None:
                    m_run, s_run = mc, sc
                else:
                    m_new = jnp.maximum(m_run, mc)
                    s_run = (s_run * jnp.exp2(m_run - m_new)
                             + sc * jnp.exp2(mc - m_new))
                    m_run = m_new
            m1 = jnp.max(m_run, axis=0, keepdims=True)
            s1 = jnp.sum(s_run * jnp.exp2(m_run - m1),
                         axis=0, keepdims=True)
            out_ref[pl.ds(g * _LG + j, 1), :] = _LN2 * (m1 + jnp.log2(s1))
        return 0

    lax.fori_loop(0, NG, lgroup, 0)


def _tc_call(zt, means, logvars, w2):
    L, Btc = zt.shape
    K = means.shape[0]
    Bb = 512
    grid = (Btc // Bb,)
    return pl.pallas_call(
        _tc_body,
        grid=grid,
        in_specs=[
            pl.BlockSpec((L, Bb), lambda i: (0, i)),
            pl.BlockSpec((K, L), lambda i: (0, 0)),
            pl.BlockSpec((K, L), lambda i: (0, 0)),
            pl.BlockSpec((K, 1), lambda i: (0, 0)),
        ],
        out_specs=pl.BlockSpec((L, Bb), lambda i: (0, i)),
        out_shape=jax.ShapeDtypeStruct((L, Btc), jnp.float32),
        scratch_shapes=[
            pltpu.VMEM((L // _LG, 3 * K, _LG), jnp.float32),
        ],
    )(zt, means, logvars, w2)


@jax.jit
def kernel(z, means, logvars, w):
    B, L = z.shape
    K = means.shape[0]
    zt = z.T
    w2 = w.reshape(K, 1)
    m, s = _sc_call(zt[:, :_B_SC], means.T.reshape(-1),
                    logvars.T.reshape(-1), w.reshape(K))
    out_tc = _tc_call(zt[:, _B_SC:], means, logvars, w2)
    out_sc = pl.pallas_call(
        _epilogue_body,
        in_specs=[
            pl.BlockSpec((L, _B_SC), lambda: (0, 0)),
            pl.BlockSpec((L, _B_SC), lambda: (0, 0)),
            pl.BlockSpec((K, 1), lambda: (0, 0)),
        ],
        out_specs=pl.BlockSpec((L, _B_SC), lambda: (0, 0)),
        out_shape=jax.ShapeDtypeStruct((L, _B_SC), jnp.float32),
    )(m, s, w2)
    return jnp.concatenate([out_sc, out_tc], axis=1).T


# speedup vs baseline: 10.0442x; 1.1529x over previous
import functools
import math

import jax
import jax.numpy as jnp
from jax import lax
from jax.experimental import pallas as pl
from jax.experimental.pallas import tpu as pltpu

_HALF_LOG_2PI = 0.5 * math.log(2.0 * math.pi)
_LOG2E = 1.4426950408889634
_LN2 = 0.6931471805599453
_LG = 8
_KC = 64


def _body(zt_ref, mu_ref, lv_ref, w_ref, out_ref, p3_s):
    K, L = mu_ref.shape
    Bb = zt_ref.shape[1]
    NG = L // _LG
    mu = mu_ref[...]
    lv = lv_ref[...]
    wv = w_ref[...]
    wmax = jnp.max(wv)
    logw = wv - wmax - jnp.log(jnp.sum(jnp.exp(wv - wmax)))
    prec = jnp.exp(-lv)
    a_all = _LOG2E * ((-_HALF_LOG_2PI) - 0.5 * lv
                      - 0.5 * prec * mu * mu + logw)
    b_all = _LOG2E * prec * mu
    c_all = (-0.5 * _LOG2E) * prec
    for g in range(NG):
        sl = slice(g * _LG, (g + 1) * _LG)
        p3_s[pl.ds(g, 1)] = jnp.concatenate(
            [a_all[:, sl], b_all[:, sl], c_all[:, sl]], axis=0)[None]

    def lgroup(g, _):
        pc = p3_s[pl.ds(g, 1)][0]
        for j in range(_LG):
            col = pc[:, j:j + 1]
            a = col[0:K]
            b = col[K:2 * K]
            c = col[2 * K:3 * K]
            zrow = zt_ref[pl.ds(g * _LG + j, 1), :]
            z2 = zrow * zrow
            m_run = None
            s_run = None
            for kc in range(K // _KC):
                ks = slice(kc * _KC, (kc + 1) * _KC)
                t2c = a[ks] + b[ks] * zrow + c[ks] * z2
                t3 = t2c.reshape(_KC // 8, 8, Bb)
                mc = jnp.max(t3, axis=0)
                sc = jnp.sum(jnp.exp2(t3 - mc[None]), axis=0)
                if m_run is None:
                    m_run, s_run = mc, sc
                else:
                    m_new = jnp.maximum(m_run, mc)
                    s_run = (s_run * jnp.exp2(m_run - m_new)
                             + sc * jnp.exp2(mc - m_new))
                    m_run = m_new
            m1 = jnp.max(m_run, axis=0, keepdims=True)
            s1 = jnp.sum(s_run * jnp.exp2(m_run - m1),
                         axis=0, keepdims=True)
            out_ref[pl.ds(g * _LG + j, 1), :] = _LN2 * (m1 + jnp.log2(s1))
        return 0

    lax.fori_loop(0, NG, lgroup, 0)


@jax.jit
def kernel(z, means, logvars, w):
    B, L = z.shape
    K = means.shape[0]
    zt = z.T
    w2 = w.reshape(K, 1)
    Bb = 512
    grid = (B // Bb,)
    out = pl.pallas_call(
        _body,
        grid=grid,
        in_specs=[
            pl.BlockSpec((L, Bb), lambda i: (0, i)),
            pl.BlockSpec((K, L), lambda i: (0, 0)),
            pl.BlockSpec((K, L), lambda i: (0, 0)),
            pl.BlockSpec((K, 1), lambda i: (0, 0)),
        ],
        out_specs=pl.BlockSpec((L, Bb), lambda i: (0, i)),
        out_shape=jax.ShapeDtypeStruct((L, B), jnp.float32),
        scratch_shapes=[
            pltpu.VMEM((L // _LG, 3 * K, _LG), jnp.float32),
        ],
    )(zt, means, logvars, w2)
    return out.T
